# baseline (speedup 1.0000x reference)
import functools

import jax
import jax.numpy as jnp
from jax import lax
from jax.experimental import pallas as pl
from jax.experimental.pallas import tpu as pltpu
from jax.experimental.pallas import tpu_sc as plsc

_N = 20000
_D = 1024
_K = 200
_EPS = 1e-08
_NP = 20480
_ROWS = 2048
_NV = _NP // 16
_CAP = 512
_IMIN = -2147483648


def _mv_body(h_ref, w_ref, b_ref, out_ref):
    out_ref[:, :] = lax.dot_general(
        w_ref[0:8, :], h_ref[:, :],
        dimension_numbers=(((1,), (1,)), ((), ())),
        preferred_element_type=jnp.float32,
    ) + b_ref[0, 0]


def _alpha_matvec(h, w_pad_t, b):
    return pl.pallas_call(
        _mv_body,
        grid=(_NP // _ROWS,),
        in_specs=[
            pl.BlockSpec((_ROWS, _D), lambda i: (i, 0)),
            pl.BlockSpec((128, _D), lambda i: (0, 0)),
            pl.BlockSpec((1, 1), lambda i: (0, 0), memory_space=pltpu.SMEM),
        ],
        out_specs=pl.BlockSpec((8, _ROWS), lambda i: (0, i)),
        out_shape=jax.ShapeDtypeStruct((8, _NP), jnp.float32),
    )(h, w_pad_t, b.reshape(1, 1))


def _keys_of(bits):
    return jnp.where(bits >= 0, bits, jnp.int32(_IMIN) - bits)


_mesh = plsc.VectorSubcoreMesh(core_axis_name="c", subcore_axis_name="s")


@functools.partial(
    pl.kernel,
    mesh=_mesh,
    compiler_params=pltpu.CompilerParams(needs_layout_passes=False),
    out_type=[
        jax.ShapeDtypeStruct((_NP,), jnp.float32),
        jax.ShapeDtypeStruct((256,), jnp.int32),
        jax.ShapeDtypeStruct((256,), jnp.int32),
        jax.ShapeDtypeStruct((256,), jnp.float32),
        jax.ShapeDtypeStruct((256,), jnp.float32),
    ],
    scratch_types=[
        pltpu.VMEM((_NP,), jnp.int32),
        pltpu.VMEM((_NP,), jnp.float32),
        pltpu.VMEM((_NP,), jnp.float32),
        pltpu.VMEM((4096,), jnp.int32),
        pltpu.VMEM((4096,), jnp.int32),
        pltpu.VMEM((_CAP,), jnp.int32),
        pltpu.VMEM((_CAP,), jnp.int32),
        pltpu.VMEM((_CAP,), jnp.int32),
        pltpu.VMEM((_CAP,), jnp.int32),
        pltpu.VMEM((256,), jnp.int32),
        pltpu.VMEM((256,), jnp.int32),
        pltpu.VMEM((256,), jnp.float32),
        pltpu.VMEM((256,), jnp.float32),
    ],
)
def _select(alpha_hbm, vols_hbm,
            w_hbm, lidx_hbm, sidx_hbm, lw_hbm, sw_hbm,
            alpha_v, vols_v, wbuf_v, histA_v, histB_v,
            ckeyL_v, cidxL_v, ckeyS_v, cidxS_v,
            lidx_v, sidx_v, lw_v, sw_v):
    wid = lax.axis_index("s") * 2 + lax.axis_index("c")

    @pl.when(wid == 0)
    def _body():
        pltpu.sync_copy(alpha_hbm, alpha_v)
        pltpu.sync_copy(vols_hbm, vols_v)
        lanes = lax.iota(jnp.int32, 16)
        ones16 = jnp.ones((16,), jnp.int32)
        zero16f = jnp.zeros((16,), jnp.float32)
        zero16i = jnp.zeros((16,), jnp.int32)

        def zero_hists(i, c):
            histA_v[pl.ds(i * 16, 16)] = zero16i
            histB_v[pl.ds(i * 16, 16)] = zero16i
            return c
        lax.fori_loop(0, 256, zero_hists, 0)

        def zero_misc(i, c):
            lidx_v[pl.ds(i * 16, 16)] = zero16i
            sidx_v[pl.ds(i * 16, 16)] = zero16i
            lw_v[pl.ds(i * 16, 16)] = zero16f
            sw_v[pl.ds(i * 16, 16)] = zero16f
            return c
        lax.fori_loop(0, 16, zero_misc, 0)

        def zero_w(i, c):
            wbuf_v[pl.ds(i * 16, 16)] = zero16f
            return c
        lax.fori_loop(0, _NV, zero_w, 0)

        def histo1(e, c):
            key = _keys_of(alpha_v[pl.ds(e * 16, 16)])
            valid = (e * 16 + lanes) < _N
            bin1 = lax.shift_right_arithmetic(key, 24) + 128
            plsc.addupdate_scatter(histA_v, [bin1 * 16 + lanes], ones16,
                                   mask=valid)
            return c
        lax.fori_loop(0, _NV, histo1, 0)

        def scan_hi(t, carry):
            acc, binv, nab = carry
            b = 255 - t
            tot = jnp.sum(histA_v[pl.ds(b * 16, 16)])
            acc2 = acc + tot
            hit = (acc2 >= _K) & (acc < _K)
            return (acc2,
                    jnp.where(hit, b, binv),
                    jnp.where(hit, acc, nab))

        def scan_lo(t, carry):
            acc, binv, nbl = carry
            tot = jnp.sum(histA_v[pl.ds(t * 16, 16)])
            acc2 = acc + tot
            hit = (acc2 >= _K) & (acc < _K)
            return (acc2,
                    jnp.where(hit, t, binv),
                    jnp.where(hit, acc, nbl))

        z = jnp.int32(0)
        _, binL1, nabL1 = lax.fori_loop(0, 256, scan_hi, (z, z, z))
        _, binS1, nblS1 = lax.fori_loop(0, 256, scan_lo, (z, z, z))

        def histo2(e, c):
            key = _keys_of(alpha_v[pl.ds(e * 16, 16)])
            valid = (e * 16 + lanes) < _N
            bin1 = lax.shift_right_arithmetic(key, 24) + 128
            bin2 = jnp.bitwise_and(lax.shift_right_arithmetic(key, 16), 255)
            hidx = bin2 * 16 + lanes
            plsc.addupdate_scatter(histA_v, [hidx], ones16,
                                   mask=valid & (bin1 == binL1))
            plsc.addupdate_scatter(histB_v, [hidx], ones16,
                                   mask=valid & (bin1 == binS1))
            return c

        def zero_histA(i, c):
            histA_v[pl.ds(i * 16, 16)] = zero16i
            return c
        lax.fori_loop(0, 256, zero_histA, 0)
        lax.fori_loop(0, _NV, histo2, 0)

        def scan2_hi(t, carry):
            acc, binv = carry
            b = 255 - t
            tot = jnp.sum(histA_v[pl.ds(b * 16, 16)])
            acc2 = acc + tot
            hit = (acc2 >= _K) & (acc < _K)
            return acc2, jnp.where(hit, b, binv)

        def scan2_lo(t, carry):
            acc, binv = carry
            tot = jnp.sum(histB_v[pl.ds(t * 16, 16)])
            acc2 = acc + tot
            hit = (acc2 >= _K) & (acc < _K)
            return acc2, jnp.where(hit, t, binv)

        _, binL2 = lax.fori_loop(0, 256, scan2_hi, (nabL1, z))
        _, binS2 = lax.fori_loop(0, 256, scan2_lo, (nblS1, z))

        thrL = jnp.bitwise_or(lax.shift_left(binL1 - 128, 8), binL2)
        thrS = jnp.bitwise_or(lax.shift_left(binS1 - 128, 8), binS2)

        def compact(e, carry):
            offL, offS = carry
            key = _keys_of(alpha_v[pl.ds(e * 16, 16)])
            gidx = e * 16 + lanes
            valid = gidx < _N
            t16 = lax.shift_right_arithmetic(key, 16)
            mL = valid & (t16 >= thrL)
            mS = valid & (t16 <= thrS)
            oL = jnp.minimum(offL, _CAP - 16)
            oS = jnp.minimum(offS, _CAP - 16)
            plsc.store_compressed(ckeyL_v.at[pl.ds(oL, 16)], key, mask=mL)
            plsc.store_compressed(cidxL_v.at[pl.ds(oL, 16)], gidx, mask=mL)
            plsc.store_compressed(ckeyS_v.at[pl.ds(oS, 16)], key, mask=mS)
            plsc.store_compressed(cidxS_v.at[pl.ds(oS, 16)], gidx, mask=mS)
            return (offL + jnp.sum(mL.astype(jnp.int32)),
                    offS + jnp.sum(mS.astype(jnp.int32)))
        nL, nS = lax.fori_loop(0, _NV, compact, (z, z))
        nL = jnp.minimum(nL, _CAP)
        nS = jnp.minimum(nS, _CAP)

        def rank_scatter(ckey_v, cidx_v, n, base, idx_out_v):
            nvreg = lax.div(n + 15, jnp.int32(16))

            def outer(iv, c):
                kv = ckey_v[pl.ds(iv * 16, 16)]
                xv = cidx_v[pl.ds(iv * 16, 16)]

                def inner(jv, rank):
                    kjv = ckey_v[pl.ds(jv * 16, 16)]
                    ijv = cidx_v[pl.ds(jv * 16, 16)]
                    for l in range(16):
                        kj = kjv[l]
                        ij = ijv[l]
                        jok = (jv * 16 + l) < n
                        gt = ((kj > kv) | ((kj == kv) & (ij < xv))) & jok
                        rank = rank + gt.astype(jnp.int32)
                    return rank
                rank = lax.fori_loop(0, nvreg, inner, zero16i)
                pos = rank - base
                msk = (pos >= 0) & (pos < _K) & ((iv * 16 + lanes) < n)
                plsc.store_scatter(idx_out_v, [pos], xv, mask=msk)
                return c
            lax.fori_loop(0, nvreg, outer, 0)

        rank_scatter(ckeyL_v, cidxL_v, nL, z, lidx_v)
        rank_scatter(ckeyS_v, cidxS_v, nS, nS - _K, sidx_v)

        def weights_for(idx_v, w_v, sign):
            def acc_loop(r, tot):
                idxv = idx_v[pl.ds(r * 16, 16)]
                volv = plsc.load_gather(vols_v, [idxv])
                wv = 1.0 / (volv + _EPS)
                wv = jnp.where((r * 16 + lanes) < _K, wv, 0.0)
                w_v[pl.ds(r * 16, 16)] = wv
                return tot + jnp.sum(wv)
            tot = lax.fori_loop(0, 13, acc_loop, jnp.float32(0.0))

            def norm_loop(r, c):
                idxv = idx_v[pl.ds(r * 16, 16)]
                wv = w_v[pl.ds(r * 16, 16)] / tot
                w_v[pl.ds(r * 16, 16)] = wv
                msk = (r * 16 + lanes) < _K
                plsc.store_scatter(wbuf_v, [idxv], sign * wv, mask=msk)
                return c
            lax.fori_loop(0, 13, norm_loop, 0)

        weights_for(lidx_v, lw_v, jnp.float32(1.0))
        weights_for(sidx_v, sw_v, jnp.float32(-1.0))

        pltpu.sync_copy(wbuf_v, w_hbm)
        pltpu.sync_copy(lidx_v, lidx_hbm)
        pltpu.sync_copy(sidx_v, sidx_hbm)
        pltpu.sync_copy(lw_v, lw_hbm)
        pltpu.sync_copy(sw_v, sw_hbm)


def kernel(h, volatilities, W, b):
    w_pad_t = jnp.zeros((128, _D), jnp.float32).at[0, :].set(W[0])
    ap = _alpha_matvec(h, w_pad_t, b)
    alpha_p = ap[0, :]
    alpha_bits = jax.lax.bitcast_convert_type(alpha_p, jnp.int32)
    vols_p = jnp.pad(volatilities, (0, _NP - _N), constant_values=1.0)
    weights_p, lidx, sidx, lw, sw = _select(alpha_bits, vols_p)
    return (alpha_p[:_N], weights_p[:_N],
            lidx[:_K], sidx[:_K], lw[:_K], sw[:_K])


# speedup vs baseline: 2.0245x; 2.0245x over previous
import functools

import jax
import jax.numpy as jnp
from jax import lax
from jax.experimental import pallas as pl
from jax.experimental.pallas import tpu as pltpu
from jax.experimental.pallas import tpu_sc as plsc

_N = 20000
_D = 1024
_K = 200
_EPS = 1e-08
_NP = 20480
_ROWS = 2048
_NW = 16
_SL = _NP // _NW
_NVW = _SL // 16
_CAPW = 128
_CAP = _NW * _CAPW
_TRASH = _CAP - 1
_IMIN = -2147483648


def _mv_body(h_ref, w_ref, b_ref, out_ref):
    out_ref[:, :] = lax.dot_general(
        w_ref[0:8, :], h_ref[:, :],
        dimension_numbers=(((1,), (1,)), ((), ())),
        preferred_element_type=jnp.float32,
    ) + b_ref[0, 0]


def _alpha_matvec(h, w_pad_t, b):
    return pl.pallas_call(
        _mv_body,
        grid=(_NP // _ROWS,),
        in_specs=[
            pl.BlockSpec((_ROWS, _D), lambda i: (i, 0)),
            pl.BlockSpec((128, _D), lambda i: (0, 0)),
            pl.BlockSpec((1, 1), lambda i: (0, 0), memory_space=pltpu.SMEM),
        ],
        out_specs=pl.BlockSpec((8, _ROWS), lambda i: (0, i)),
        out_shape=jax.ShapeDtypeStruct((8, _NP), jnp.float32),
    )(h, w_pad_t, b.reshape(1, 1))


def _keys_of(bits):
    return jnp.where(bits >= 0, bits, jnp.int32(_IMIN) - bits)


_mesh = plsc.VectorSubcoreMesh(core_axis_name="c", subcore_axis_name="s")


@functools.partial(
    pl.kernel,
    mesh=_mesh,
    compiler_params=pltpu.CompilerParams(needs_layout_passes=False),
    out_type=[
        jax.ShapeDtypeStruct((_NP,), jnp.float32),
        jax.ShapeDtypeStruct((256,), jnp.int32),
        jax.ShapeDtypeStruct((256,), jnp.int32),
        jax.ShapeDtypeStruct((256,), jnp.float32),
        jax.ShapeDtypeStruct((256,), jnp.float32),
    ],
    scratch_types=[
        pltpu.VMEM((_SL,), jnp.int32),
        pltpu.VMEM((_NP,), jnp.float32),
        pltpu.VMEM((_NP,), jnp.float32),
        pltpu.VMEM((4096,), jnp.int32),
        pltpu.VMEM((4096,), jnp.int32),
        pltpu.VMEM((_CAPW,), jnp.int32),
        pltpu.VMEM((_CAPW,), jnp.int32),
        pltpu.VMEM((_CAPW,), jnp.int32),
        pltpu.VMEM((_CAPW,), jnp.int32),
        pltpu.VMEM((_CAPW,), jnp.int32),
        pltpu.VMEM((_CAPW,), jnp.int32),
        pltpu.VMEM((_CAP,), jnp.int32),
        pltpu.VMEM((_CAP,), jnp.int32),
        pltpu.VMEM((_CAP,), jnp.int32),
        pltpu.VMEM((_CAP,), jnp.int32),
        pltpu.VMEM((256,), jnp.int32),
        pltpu.VMEM((256,), jnp.int32),
        pltpu.VMEM((256,), jnp.int32),
        pltpu.VMEM((256,), jnp.float32),
        pltpu.VMEM((256,), jnp.float32),
        pltpu.VMEM((4096,), jnp.int32),
        pltpu.VMEM((256,), jnp.int32),
        pltpu.VMEM_SHARED((4096,), jnp.int32),
        pltpu.VMEM_SHARED((4096,), jnp.int32),
        pltpu.VMEM_SHARED((4096,), jnp.int32),
        pltpu.VMEM_SHARED((_CAP,), jnp.int32),
        pltpu.VMEM_SHARED((_CAP,), jnp.int32),
        pltpu.VMEM_SHARED((_CAP,), jnp.int32),
        pltpu.VMEM_SHARED((_CAP,), jnp.int32),
        pltpu.VMEM_SHARED((256,), jnp.int32),
        pltpu.VMEM_SHARED((256,), jnp.int32),
        pltpu.VMEM_SHARED((256,), jnp.int32),
    ],
)
def _select(bits_hbm, vols_hbm,
            w_hbm, lidx_hbm, sidx_hbm, lw_hbm, sw_hbm,
            abits_v, vols_v, wbuf_v, histA_v, histB_v,
            ckeyL_v, cidxL_v, ckeyS_v, cidxS_v,
            scatL_v, scatS_v,
            mkeyL_v, midxL_v, mkeyS_v, midxS_v,
            cnt_v, lidx_v, sidx_v, lw_v, sw_v,
            iota4k_v, iota256_v,
            histA_sh, histBL_sh, histBS_sh,
            ckeyL_sh, cidxL_sh, ckeyS_sh, cidxS_sh,
            cnt_sh, lidx_sh, sidx_sh):
    core = lax.axis_index("c")
    sid = lax.axis_index("s")

    @pl.when(core == 0)
    def _body():
        base = sid * _SL
        pltpu.sync_copy(bits_hbm.at[pl.ds(base, _SL)], abits_v)
        lanes = lax.iota(jnp.int32, 16)
        ones16 = jnp.ones((16,), jnp.int32)
        zero16f = jnp.zeros((16,), jnp.float32)
        zero16i = jnp.zeros((16,), jnp.int32)
        z = jnp.int32(0)

        def zero_hists(i, c):
            histA_v[pl.ds(i * 16, 16)] = zero16i
            histB_v[pl.ds(i * 16, 16)] = zero16i
            iota4k_v[pl.ds(i * 16, 16)] = i * 16 + lanes
            return c
        lax.fori_loop(0, 256, zero_hists, 0)

        def fill_iota256(i, c):
            iota256_v[pl.ds(i * 16, 16)] = i * 16 + lanes
            return c
        lax.fori_loop(0, 16, fill_iota256, 0)

        @pl.when(sid == 0)
        def _z():
            pltpu.sync_copy(vols_hbm, vols_v)
            pltpu.sync_copy(histA_v, histA_sh)
            pltpu.sync_copy(histA_v, histBL_sh)
            pltpu.sync_copy(histA_v, histBS_sh)

            def zero_misc(i, c):
                lidx_v[pl.ds(i * 16, 16)] = zero16i
                return c
            lax.fori_loop(0, 16, zero_misc, 0)
            pltpu.sync_copy(lidx_v, lidx_sh)
            pltpu.sync_copy(lidx_v, sidx_sh)
        plsc.subcore_barrier()

        def histo1(e, c):
            key = _keys_of(abits_v[pl.ds(e * 16, 16)])
            valid = (base + e * 16 + lanes) < _N
            bin1 = lax.shift_right_arithmetic(key, 24) + 128
            plsc.addupdate_scatter(histA_v, [bin1 * 16 + lanes], ones16,
                                   mask=valid)
            return c
        lax.fori_loop(0, _NVW, histo1, 0)
        pltpu.sync_copy(histA_v, histA_sh.at[iota4k_v], add=True)
        plsc.subcore_barrier()

        pltpu.sync_copy(histA_sh, histA_v)

        def scan_hi(t, carry):
            acc, binv, nab = carry
            b = 255 - t
            tot = jnp.sum(histA_v[pl.ds(b * 16, 16)])
            acc2 = acc + tot
            hit = (acc2 >= _K) & (acc < _K)
            return (acc2,
                    jnp.where(hit, b, binv),
                    jnp.where(hit, acc, nab))

        def scan_lo(t, carry):
            acc, binv, nbl = carry
            tot = jnp.sum(histA_v[pl.ds(t * 16, 16)])
            acc2 = acc + tot
            hit = (acc2 >= _K) & (acc < _K)
            return (acc2,
                    jnp.where(hit, t, binv),
                    jnp.where(hit, acc, nbl))

        _, binL1, nabL1 = lax.fori_loop(0, 256, scan_hi, (z, z, z))
        _, binS1, nblS1 = lax.fori_loop(0, 256, scan_lo, (z, z, z))

        def zero_again(i, c):
            histA_v[pl.ds(i * 16, 16)] = zero16i
            histB_v[pl.ds(i * 16, 16)] = zero16i
            return c
        lax.fori_loop(0, 256, zero_again, 0)

        def histo2(e, c):
            key = _keys_of(abits_v[pl.ds(e * 16, 16)])
            valid = (base + e * 16 + lanes) < _N
            bin1 = lax.shift_right_arithmetic(key, 24) + 128
            bin2 = jnp.bitwise_and(lax.shift_right_arithmetic(key, 16), 255)
            hidx = bin2 * 16 + lanes
            plsc.addupdate_scatter(histA_v, [hidx], ones16,
                                   mask=valid & (bin1 == binL1))
            plsc.addupdate_scatter(histB_v, [hidx], ones16,
                                   mask=valid & (bin1 == binS1))
            return c
        lax.fori_loop(0, _NVW, histo2, 0)
        pltpu.sync_copy(histA_v, histBL_sh.at[iota4k_v], add=True)
        pltpu.sync_copy(histB_v, histBS_sh.at[iota4k_v], add=True)
        plsc.subcore_barrier()

        pltpu.sync_copy(histBL_sh, histA_v)
        pltpu.sync_copy(histBS_sh, histB_v)

        def scan2_hi(t, carry):
            acc, binv = carry
            b = 255 - t
            tot = jnp.sum(histA_v[pl.ds(b * 16, 16)])
            acc2 = acc + tot
            hit = (acc2 >= _K) & (acc < _K)
            return acc2, jnp.where(hit, b, binv)

        def scan2_lo(t, carry):
            acc, binv = carry
            tot = jnp.sum(histB_v[pl.ds(t * 16, 16)])
            acc2 = acc + tot
            hit = (acc2 >= _K) & (acc < _K)
            return acc2, jnp.where(hit, t, binv)

        _, binL2 = lax.fori_loop(0, 256, scan2_hi, (nabL1, z))
        _, binS2 = lax.fori_loop(0, 256, scan2_lo, (nblS1, z))

        thrL = jnp.bitwise_or(lax.shift_left(binL1 - 128, 8), binL2)
        thrS = jnp.bitwise_or(lax.shift_left(binS1 - 128, 8), binS2)

        def compact(e, carry):
            offL, offS = carry
            key = _keys_of(abits_v[pl.ds(e * 16, 16)])
            gidx = base + e * 16 + lanes
            valid = gidx < _N
            t16 = lax.shift_right_arithmetic(key, 16)
            mL = valid & (t16 >= thrL)
            mS = valid & (t16 <= thrS)
            oL = jnp.minimum(offL, _CAPW - 16)
            oS = jnp.minimum(offS, _CAPW - 16)
            plsc.store_compressed(ckeyL_v.at[pl.ds(oL, 16)], key, mask=mL)
            plsc.store_compressed(cidxL_v.at[pl.ds(oL, 16)], gidx, mask=mL)
            plsc.store_compressed(ckeyS_v.at[pl.ds(oS, 16)], key, mask=mS)
            plsc.store_compressed(cidxS_v.at[pl.ds(oS, 16)], gidx, mask=mS)
            return (offL + jnp.sum(mL.astype(jnp.int32)),
                    offS + jnp.sum(mS.astype(jnp.int32)))
        cL, cS = lax.fori_loop(0, _NVW, compact, (z, z))
        cL = jnp.minimum(cL, _CAPW)
        cS = jnp.minimum(cS, _CAPW)
        cnt_vec = jnp.where(lanes == 0, cL, jnp.where(lanes == 1, cS, 0))
        cnt_v[pl.ds(0, 16)] = cnt_vec
        pltpu.sync_copy(cnt_v.at[pl.ds(0, 16)],
                        cnt_sh.at[pl.ds(sid * 16, 16)])
        plsc.subcore_barrier()

        pltpu.sync_copy(cnt_sh, cnt_v)
        offL = z
        offS = z
        myoffL = z
        myoffS = z
        for w in range(_NW):
            row = cnt_v[pl.ds(w * 16, 16)]
            cLw = row[0]
            cSw = row[1]
            myoffL = jnp.where(sid == w, offL, myoffL)
            myoffS = jnp.where(sid == w, offS, myoffS)
            offL = offL + cLw
            offS = offS + cSw
        nL = offL
        nS = offS

        def mkscat(e, c):
            ln = e * 16 + lanes
            tL = jnp.where(ln < cL, myoffL + ln, jnp.int32(_TRASH))
            tS = jnp.where(ln < cS, myoffS + ln, jnp.int32(_TRASH))
            scatL_v[pl.ds(e * 16, 16)] = tL
            scatS_v[pl.ds(e * 16, 16)] = tS
            return c
        lax.fori_loop(0, _CAPW // 16, mkscat, 0)
        pltpu.sync_copy(ckeyL_v, ckeyL_sh.at[scatL_v])
        pltpu.sync_copy(cidxL_v, cidxL_sh.at[scatL_v])
        pltpu.sync_copy(ckeyS_v, ckeyS_sh.at[scatS_v])
        pltpu.sync_copy(cidxS_v, cidxS_sh.at[scatS_v])
        plsc.subcore_barrier()

        pltpu.sync_copy(ckeyL_sh, mkeyL_v)
        pltpu.sync_copy(cidxL_sh, midxL_v)
        pltpu.sync_copy(ckeyS_sh, mkeyS_v)
        pltpu.sync_copy(cidxS_sh, midxS_v)

        def zero_out(i, c):
            lidx_v[pl.ds(i * 16, 16)] = zero16i
            sidx_v[pl.ds(i * 16, 16)] = zero16i
            return c
        lax.fori_loop(0, 16, zero_out, 0)

        def rank_side(ckey_v, cidx_v, n, rbase, idx_out_v):
            nvreg = lax.div(n + 15, jnp.int32(16))

            def do_iv(iv):
                kv = ckey_v[pl.ds(iv * 16, 16)]
                xv = cidx_v[pl.ds(iv * 16, 16)]

                def inner(jv, rank):
                    kjv = ckey_v[pl.ds(jv * 16, 16)]
                    ijv = cidx_v[pl.ds(jv * 16, 16)]
                    for l in range(16):
                        kj = kjv[l]
                        ij = ijv[l]
                        jok = (jv * 16 + l) < n
                        gt = ((kj > kv) | ((kj == kv) & (ij < xv))) & jok
                        rank = rank + gt.astype(jnp.int32)
                    return rank
                rank = lax.fori_loop(0, nvreg, inner, zero16i)
                pos = rank - rbase
                msk = (pos >= 0) & (pos < _K) & ((iv * 16 + lanes) < n)
                plsc.store_scatter(idx_out_v, [pos], xv, mask=msk)

            @pl.when(sid * 16 < n)
            def _a():
                do_iv(sid)

            @pl.when((sid + _NW) * 16 < n)
            def _b():
                do_iv(sid + _NW)

        rank_side(mkeyL_v, midxL_v, nL, z, lidx_v)
        rank_side(mkeyS_v, midxS_v, nS, nS - _K, sidx_v)
        pltpu.sync_copy(lidx_v, lidx_sh.at[iota256_v], add=True)
        pltpu.sync_copy(sidx_v, sidx_sh.at[iota256_v], add=True)
        plsc.subcore_barrier()

        @pl.when(sid == 0)
        def _fin():
            pltpu.sync_copy(lidx_sh, lidx_v)
            pltpu.sync_copy(sidx_sh, sidx_v)

            def zero_w(i, c):
                wbuf_v[pl.ds(i * 16, 16)] = zero16f
                return c
            lax.fori_loop(0, _NP // 16, zero_w, 0)

            def weights_for(idx_v, w_v, sign):
                def acc_loop(r, tot):
                    idxv = idx_v[pl.ds(r * 16, 16)]
                    volv = plsc.load_gather(vols_v, [idxv])
                    wv = 1.0 / (volv + _EPS)
                    wv = jnp.where((r * 16 + lanes) < _K, wv, 0.0)
                    w_v[pl.ds(r * 16, 16)] = wv
                    return tot + jnp.sum(wv)
                tot = lax.fori_loop(0, 13, acc_loop, jnp.float32(0.0))

                def norm_loop(r, c):
                    idxv = idx_v[pl.ds(r * 16, 16)]
                    wv = w_v[pl.ds(r * 16, 16)] / tot
                    w_v[pl.ds(r * 16, 16)] = wv
                    msk = (r * 16 + lanes) < _K
                    plsc.store_scatter(wbuf_v, [idxv], sign * wv, mask=msk)
                    return c
                lax.fori_loop(0, 13, norm_loop, 0)

            weights_for(lidx_v, lw_v, jnp.float32(1.0))
            weights_for(sidx_v, sw_v, jnp.float32(-1.0))

            pltpu.sync_copy(wbuf_v, w_hbm)
            pltpu.sync_copy(lidx_v, lidx_hbm)
            pltpu.sync_copy(sidx_v, sidx_hbm)
            pltpu.sync_copy(lw_v, lw_hbm)
            pltpu.sync_copy(sw_v, sw_hbm)


def kernel(h, volatilities, W, b):
    w_pad_t = jnp.zeros((128, _D), jnp.float32).at[0, :].set(W[0])
    ap = _alpha_matvec(h, w_pad_t, b)
    alpha_p = ap[0, :]
    alpha_bits = jax.lax.bitcast_convert_type(alpha_p, jnp.int32)
    vols_p = jnp.pad(volatilities, (0, _NP - _N), constant_values=1.0)
    weights_p, lidx, sidx, lw, sw = _select(alpha_bits, vols_p)
    return (alpha_p[:_N], weights_p[:_N],
            lidx[:_K], sidx[:_K], lw[:_K], sw[:_K])


# speedup vs baseline: 2.0769x; 1.0259x over previous
import functools

import jax
import jax.numpy as jnp
from jax import lax
from jax.experimental import pallas as pl
from jax.experimental.pallas import tpu as pltpu
from jax.experimental.pallas import tpu_sc as plsc

_N = 20000
_D = 1024
_K = 200
_EPS = 1e-08
_NP = 20480
_ROWS = 2048
_NW = 16
_SL = _NP // _NW
_NVW = _SL // 16
_CAPW = 128
_CAP = _NW * _CAPW
_TRASH = _CAP - 1
_IMIN = -2147483648


def _mv_body(h_ref, w_ref, b_ref, alpha_ref, bits_ref):
    r = lax.dot_general(
        w_ref[0:8, :], h_ref[:, :],
        dimension_numbers=(((1,), (1,)), ((), ())),
        preferred_element_type=jnp.float32,
    ) + b_ref[0, 0]
    row = r[0:1, :]
    alpha_ref[:, :] = row
    bits_ref[:, :] = jax.lax.bitcast_convert_type(row, jnp.int32)


def _alpha_matvec(h, w_pad_t, b):
    return pl.pallas_call(
        _mv_body,
        grid=(_NP // _ROWS,),
        in_specs=[
            pl.BlockSpec((_ROWS, _D), lambda i: (i, 0)),
            pl.BlockSpec((128, _D), lambda i: (0, 0)),
            pl.BlockSpec((1, 1), lambda i: (0, 0), memory_space=pltpu.SMEM),
        ],
        out_specs=[
            pl.BlockSpec((1, _ROWS), lambda i: (0, i)),
            pl.BlockSpec((1, _ROWS), lambda i: (0, i)),
        ],
        out_shape=[
            jax.ShapeDtypeStruct((1, _N), jnp.float32),
            jax.ShapeDtypeStruct((1, _NP), jnp.int32),
        ],
    )(h, w_pad_t, b.reshape(1, 1))


def _keys_of(bits):
    return jnp.where(bits >= 0, bits, jnp.int32(_IMIN) - bits)


_mesh = plsc.VectorSubcoreMesh(core_axis_name="c", subcore_axis_name="s")


@functools.partial(
    pl.kernel,
    mesh=_mesh,
    compiler_params=pltpu.CompilerParams(needs_layout_passes=False),
    out_type=[
        jax.ShapeDtypeStruct((_N,), jnp.float32),
        jax.ShapeDtypeStruct((256,), jnp.int32),
        jax.ShapeDtypeStruct((256,), jnp.int32),
        jax.ShapeDtypeStruct((256,), jnp.float32),
        jax.ShapeDtypeStruct((256,), jnp.float32),
    ],
    scratch_types=[
        pltpu.VMEM((_SL,), jnp.int32),
        pltpu.VMEM((_NP,), jnp.float32),
        pltpu.VMEM((_NP,), jnp.float32),
        pltpu.VMEM((4096,), jnp.int32),
        pltpu.VMEM((4096,), jnp.int32),
        pltpu.VMEM((_CAPW,), jnp.int32),
        pltpu.VMEM((_CAPW,), jnp.int32),
        pltpu.VMEM((_CAPW,), jnp.int32),
        pltpu.VMEM((_CAPW,), jnp.int32),
        pltpu.VMEM((_CAPW,), jnp.int32),
        pltpu.VMEM((_CAPW,), jnp.int32),
        pltpu.VMEM((_CAP,), jnp.int32),
        pltpu.VMEM((_CAP,), jnp.int32),
        pltpu.VMEM((_CAP,), jnp.int32),
        pltpu.VMEM((_CAP,), jnp.int32),
        pltpu.VMEM((256,), jnp.int32),
        pltpu.VMEM((256,), jnp.int32),
        pltpu.VMEM((256,), jnp.int32),
        pltpu.VMEM((256,), jnp.float32),
        pltpu.VMEM((256,), jnp.float32),
        pltpu.VMEM((4096,), jnp.int32),
        pltpu.VMEM((256,), jnp.int32),
        pltpu.VMEM_SHARED((4096,), jnp.int32),
        pltpu.VMEM_SHARED((4096,), jnp.int32),
        pltpu.VMEM_SHARED((4096,), jnp.int32),
        pltpu.VMEM_SHARED((_CAP,), jnp.int32),
        pltpu.VMEM_SHARED((_CAP,), jnp.int32),
        pltpu.VMEM_SHARED((_CAP,), jnp.int32),
        pltpu.VMEM_SHARED((_CAP,), jnp.int32),
        pltpu.VMEM_SHARED((256,), jnp.int32),
        pltpu.VMEM_SHARED((256,), jnp.int32),
        pltpu.VMEM_SHARED((256,), jnp.int32),
    ],
)
def _select(bits_hbm, vols_hbm,
            w_hbm, lidx_hbm, sidx_hbm, lw_hbm, sw_hbm,
            abits_v, vols_v, wbuf_v, histA_v, histB_v,
            ckeyL_v, cidxL_v, ckeyS_v, cidxS_v,
            scatL_v, scatS_v,
            mkeyL_v, midxL_v, mkeyS_v, midxS_v,
            cnt_v, lidx_v, sidx_v, lw_v, sw_v,
            iota4k_v, iota256_v,
            histA_sh, histBL_sh, histBS_sh,
            ckeyL_sh, cidxL_sh, ckeyS_sh, cidxS_sh,
            cnt_sh, lidx_sh, sidx_sh):
    core = lax.axis_index("c")
    sid = lax.axis_index("s")

    @pl.when(core == 0)
    def _body():
        base = sid * _SL
        pltpu.sync_copy(bits_hbm.at[pl.ds(base, _SL)], abits_v)
        lanes = lax.iota(jnp.int32, 16)
        ones16 = jnp.ones((16,), jnp.int32)
        zero16f = jnp.zeros((16,), jnp.float32)
        zero16i = jnp.zeros((16,), jnp.int32)
        z = jnp.int32(0)

        def zero_hists(i, c):
            histA_v[pl.ds(i * 16, 16)] = zero16i
            histB_v[pl.ds(i * 16, 16)] = zero16i
            iota4k_v[pl.ds(i * 16, 16)] = i * 16 + lanes
            return c
        lax.fori_loop(0, 256, zero_hists, 0)

        def fill_iota256(i, c):
            iota256_v[pl.ds(i * 16, 16)] = i * 16 + lanes
            return c
        lax.fori_loop(0, 16, fill_iota256, 0)

        @pl.when(sid == 0)
        def _z():
            pltpu.sync_copy(vols_hbm, vols_v.at[pl.ds(0, _N)])
            pltpu.sync_copy(histA_v, histA_sh)
            pltpu.sync_copy(histA_v, histBL_sh)
            pltpu.sync_copy(histA_v, histBS_sh)

            def zero_misc(i, c):
                lidx_v[pl.ds(i * 16, 16)] = zero16i
                return c
            lax.fori_loop(0, 16, zero_misc, 0)
            pltpu.sync_copy(lidx_v, lidx_sh)
            pltpu.sync_copy(lidx_v, sidx_sh)
        plsc.subcore_barrier()

        def histo1(e, c):
            key = _keys_of(abits_v[pl.ds(e * 16, 16)])
            valid = (base + e * 16 + lanes) < _N
            bin1 = lax.shift_right_arithmetic(key, 24) + 128
            plsc.addupdate_scatter(histA_v, [bin1 * 16 + lanes], ones16,
                                   mask=valid)
            return c
        lax.fori_loop(0, _NVW, histo1, 0)
        pltpu.sync_copy(histA_v, histA_sh.at[iota4k_v], add=True)
        plsc.subcore_barrier()

        pltpu.sync_copy(histA_sh, histA_v)

        def scan_hi(t, carry):
            acc, binv, nab = carry
            b = 255 - t
            tot = jnp.sum(histA_v[pl.ds(b * 16, 16)])
            acc2 = acc + tot
            hit = (acc2 >= _K) & (acc < _K)
            return (acc2,
                    jnp.where(hit, b, binv),
                    jnp.where(hit, acc, nab))

        def scan_lo(t, carry):
            acc, binv, nbl = carry
            tot = jnp.sum(histA_v[pl.ds(t * 16, 16)])
            acc2 = acc + tot
            hit = (acc2 >= _K) & (acc < _K)
            return (acc2,
                    jnp.where(hit, t, binv),
                    jnp.where(hit, acc, nbl))

        _, binL1, nabL1 = lax.fori_loop(0, 256, scan_hi, (z, z, z))
        _, binS1, nblS1 = lax.fori_loop(0, 256, scan_lo, (z, z, z))

        def zero_again(i, c):
            histA_v[pl.ds(i * 16, 16)] = zero16i
            histB_v[pl.ds(i * 16, 16)] = zero16i
            return c
        lax.fori_loop(0, 256, zero_again, 0)

        def histo2(e, c):
            key = _keys_of(abits_v[pl.ds(e * 16, 16)])
            valid = (base + e * 16 + lanes) < _N
            bin1 = lax.shift_right_arithmetic(key, 24) + 128
            bin2 = jnp.bitwise_and(lax.shift_right_arithmetic(key, 16), 255)
            hidx = bin2 * 16 + lanes
            plsc.addupdate_scatter(histA_v, [hidx], ones16,
                                   mask=valid & (bin1 == binL1))
            plsc.addupdate_scatter(histB_v, [hidx], ones16,
                                   mask=valid & (bin1 == binS1))
            return c
        lax.fori_loop(0, _NVW, histo2, 0)
        pltpu.sync_copy(histA_v, histBL_sh.at[iota4k_v], add=True)
        pltpu.sync_copy(histB_v, histBS_sh.at[iota4k_v], add=True)
        plsc.subcore_barrier()

        pltpu.sync_copy(histBL_sh, histA_v)
        pltpu.sync_copy(histBS_sh, histB_v)

        def scan2_hi(t, carry):
            acc, binv = carry
            b = 255 - t
            tot = jnp.sum(histA_v[pl.ds(b * 16, 16)])
            acc2 = acc + tot
            hit = (acc2 >= _K) & (acc < _K)
            return acc2, jnp.where(hit, b, binv)

        def scan2_lo(t, carry):
            acc, binv = carry
            tot = jnp.sum(histB_v[pl.ds(t * 16, 16)])
            acc2 = acc + tot
            hit = (acc2 >= _K) & (acc < _K)
            return acc2, jnp.where(hit, t, binv)

        _, binL2 = lax.fori_loop(0, 256, scan2_hi, (nabL1, z))
        _, binS2 = lax.fori_loop(0, 256, scan2_lo, (nblS1, z))

        thrL = jnp.bitwise_or(lax.shift_left(binL1 - 128, 8), binL2)
        thrS = jnp.bitwise_or(lax.shift_left(binS1 - 128, 8), binS2)

        def compact(e, carry):
            offL, offS = carry
            key = _keys_of(abits_v[pl.ds(e * 16, 16)])
            gidx = base + e * 16 + lanes
            valid = gidx < _N
            t16 = lax.shift_right_arithmetic(key, 16)
            mL = valid & (t16 >= thrL)
            mS = valid & (t16 <= thrS)
            oL = jnp.minimum(offL, _CAPW - 16)
            oS = jnp.minimum(offS, _CAPW - 16)
            plsc.store_compressed(ckeyL_v.at[pl.ds(oL, 16)], key, mask=mL)
            plsc.store_compressed(cidxL_v.at[pl.ds(oL, 16)], gidx, mask=mL)
            plsc.store_compressed(ckeyS_v.at[pl.ds(oS, 16)], key, mask=mS)
            plsc.store_compressed(cidxS_v.at[pl.ds(oS, 16)], gidx, mask=mS)
            return (offL + jnp.sum(mL.astype(jnp.int32)),
                    offS + jnp.sum(mS.astype(jnp.int32)))
        cL, cS = lax.fori_loop(0, _NVW, compact, (z, z))
        cL = jnp.minimum(cL, _CAPW)
        cS = jnp.minimum(cS, _CAPW)
        cnt_vec = jnp.where(lanes == 0, cL, jnp.where(lanes == 1, cS, 0))
        cnt_v[pl.ds(0, 16)] = cnt_vec
        pltpu.sync_copy(cnt_v.at[pl.ds(0, 16)],
                        cnt_sh.at[pl.ds(sid * 16, 16)])
        plsc.subcore_barrier()

        pltpu.sync_copy(cnt_sh, cnt_v)
        offL = z
        offS = z
        myoffL = z
        myoffS = z
        for w in range(_NW):
            row = cnt_v[pl.ds(w * 16, 16)]
            cLw = row[0]
            cSw = row[1]
            myoffL = jnp.where(sid == w, offL, myoffL)
            myoffS = jnp.where(sid == w, offS, myoffS)
            offL = offL + cLw
            offS = offS + cSw
        nL = offL
        nS = offS

        def mkscat(e, c):
            ln = e * 16 + lanes
            tL = jnp.where(ln < cL, myoffL + ln, jnp.int32(_TRASH))
            tS = jnp.where(ln < cS, myoffS + ln, jnp.int32(_TRASH))
            scatL_v[pl.ds(e * 16, 16)] = tL
            scatS_v[pl.ds(e * 16, 16)] = tS
            return c
        lax.fori_loop(0, _CAPW // 16, mkscat, 0)
        pltpu.sync_copy(ckeyL_v, ckeyL_sh.at[scatL_v])
        pltpu.sync_copy(cidxL_v, cidxL_sh.at[scatL_v])
        pltpu.sync_copy(ckeyS_v, ckeyS_sh.at[scatS_v])
        pltpu.sync_copy(cidxS_v, cidxS_sh.at[scatS_v])
        plsc.subcore_barrier()

        pltpu.sync_copy(ckeyL_sh, mkeyL_v)
        pltpu.sync_copy(cidxL_sh, midxL_v)
        pltpu.sync_copy(ckeyS_sh, mkeyS_v)
        pltpu.sync_copy(cidxS_sh, midxS_v)

        def zero_out(i, c):
            lidx_v[pl.ds(i * 16, 16)] = zero16i
            sidx_v[pl.ds(i * 16, 16)] = zero16i
            return c
        lax.fori_loop(0, 16, zero_out, 0)

        def rank_side(ckey_v, cidx_v, n, rbase, idx_out_v):
            nvreg = lax.div(n + 15, jnp.int32(16))

            def do_iv(iv):
                kv = ckey_v[pl.ds(iv * 16, 16)]
                xv = cidx_v[pl.ds(iv * 16, 16)]

                def inner(jv, rank):
                    kjv = ckey_v[pl.ds(jv * 16, 16)]
                    ijv = cidx_v[pl.ds(jv * 16, 16)]
                    for l in range(16):
                        kj = kjv[l]
                        ij = ijv[l]
                        jok = (jv * 16 + l) < n
                        gt = ((kj > kv) | ((kj == kv) & (ij < xv))) & jok
                        rank = rank + gt.astype(jnp.int32)
                    return rank
                rank = lax.fori_loop(0, nvreg, inner, zero16i)
                pos = rank - rbase
                msk = (pos >= 0) & (pos < _K) & ((iv * 16 + lanes) < n)
                plsc.store_scatter(idx_out_v, [pos], xv, mask=msk)

            @pl.when(sid * 16 < n)
            def _a():
                do_iv(sid)

            @pl.when((sid + _NW) * 16 < n)
            def _b():
                do_iv(sid + _NW)

        rank_side(mkeyL_v, midxL_v, nL, z, lidx_v)
        rank_side(mkeyS_v, midxS_v, nS, nS - _K, sidx_v)
        pltpu.sync_copy(lidx_v, lidx_sh.at[iota256_v], add=True)
        pltpu.sync_copy(sidx_v, sidx_sh.at[iota256_v], add=True)
        plsc.subcore_barrier()

        @pl.when(sid == 0)
        def _fin():
            pltpu.sync_copy(lidx_sh, lidx_v)
            pltpu.sync_copy(sidx_sh, sidx_v)

            def zero_w(i, c):
                wbuf_v[pl.ds(i * 16, 16)] = zero16f
                return c
            lax.fori_loop(0, _NP // 16, zero_w, 0)

            def weights_for(idx_v, w_v, sign):
                def acc_loop(r, tot):
                    idxv = idx_v[pl.ds(r * 16, 16)]
                    volv = plsc.load_gather(vols_v, [idxv])
                    wv = 1.0 / (volv + _EPS)
                    wv = jnp.where((r * 16 + lanes) < _K, wv, 0.0)
                    w_v[pl.ds(r * 16, 16)] = wv
                    return tot + jnp.sum(wv)
                tot = lax.fori_loop(0, 13, acc_loop, jnp.float32(0.0))

                def norm_loop(r, c):
                    idxv = idx_v[pl.ds(r * 16, 16)]
                    wv = w_v[pl.ds(r * 16, 16)] / tot
                    w_v[pl.ds(r * 16, 16)] = wv
                    msk = (r * 16 + lanes) < _K
                    plsc.store_scatter(wbuf_v, [idxv], sign * wv, mask=msk)
                    return c
                lax.fori_loop(0, 13, norm_loop, 0)

            weights_for(lidx_v, lw_v, jnp.float32(1.0))
            weights_for(sidx_v, sw_v, jnp.float32(-1.0))

            pltpu.sync_copy(wbuf_v.at[pl.ds(0, _N)], w_hbm)
            pltpu.sync_copy(lidx_v, lidx_hbm)
            pltpu.sync_copy(sidx_v, sidx_hbm)
            pltpu.sync_copy(lw_v, lw_hbm)
            pltpu.sync_copy(sw_v, sw_hbm)


def kernel(h, volatilities, W, b):
    w_pad_t = jnp.zeros((128, _D), jnp.float32).at[0, :].set(W[0])
    alpha, bits = _alpha_matvec(h, w_pad_t, b)
    weights, lidx, sidx, lw, sw = _select(bits.reshape(_NP), volatilities)
    return (alpha.reshape(_N), weights,
            lidx[:_K], sidx[:_K], lw[:_K], sw[:_K])


# speedup vs baseline: 2.2116x; 1.0648x over previous
import functools

import jax
import jax.numpy as jnp
from jax import lax
from jax.experimental import pallas as pl
from jax.experimental.pallas import tpu as pltpu
from jax.experimental.pallas import tpu_sc as plsc

_N = 20000
_D = 1024
_K = 200
_EPS = 1e-08
_NP = 20480
_ROWS = 2048
_NW = 16
_SL = _NP // _NW
_NVW = _SL // 16
_CAPW = 128
_CAP = _NW * _CAPW
_TRASH = _CAP - 1
_IMIN = -2147483648


def _mv_body(h_ref, w_ref, b_ref, alpha_ref, bits_ref):
    r = lax.dot_general(
        w_ref[0:8, :], h_ref[:, :],
        dimension_numbers=(((1,), (1,)), ((), ())),
        preferred_element_type=jnp.float32,
    ) + b_ref[0, 0]
    row = r[0:1, :]
    alpha_ref[:, :] = row
    bits_ref[:, :] = jax.lax.bitcast_convert_type(row, jnp.int32)


def _alpha_matvec(h, w_pad_t, b):
    return pl.pallas_call(
        _mv_body,
        grid=(_NP // _ROWS,),
        in_specs=[
            pl.BlockSpec((_ROWS, _D), lambda i: (i, 0)),
            pl.BlockSpec((128, _D), lambda i: (0, 0)),
            pl.BlockSpec((1, 1), lambda i: (0, 0), memory_space=pltpu.SMEM),
        ],
        out_specs=[
            pl.BlockSpec((1, _ROWS), lambda i: (0, i)),
            pl.BlockSpec((1, _ROWS), lambda i: (0, i)),
        ],
        out_shape=[
            jax.ShapeDtypeStruct((1, _N), jnp.float32),
            jax.ShapeDtypeStruct((1, _NP), jnp.int32),
        ],
    )(h, w_pad_t, b.reshape(1, 1))


def _keys_of(bits):
    return jnp.where(bits >= 0, bits, jnp.int32(_IMIN) - bits)


_mesh = plsc.VectorSubcoreMesh(core_axis_name="c", subcore_axis_name="s")


@functools.partial(
    pl.kernel,
    mesh=_mesh,
    compiler_params=pltpu.CompilerParams(needs_layout_passes=False),
    out_type=[
        jax.ShapeDtypeStruct((_N,), jnp.float32),
        jax.ShapeDtypeStruct((256,), jnp.int32),
        jax.ShapeDtypeStruct((256,), jnp.int32),
        jax.ShapeDtypeStruct((256,), jnp.float32),
        jax.ShapeDtypeStruct((256,), jnp.float32),
    ],
    scratch_types=[
        pltpu.VMEM((_SL,), jnp.int32),
        pltpu.VMEM((_NP,), jnp.float32),
        pltpu.VMEM((_NP,), jnp.float32),
        pltpu.VMEM((4096,), jnp.int32),
        pltpu.VMEM((4096,), jnp.int32),
        pltpu.VMEM((_CAPW,), jnp.int32),
        pltpu.VMEM((_CAPW,), jnp.int32),
        pltpu.VMEM((_CAPW,), jnp.int32),
        pltpu.VMEM((_CAPW,), jnp.int32),
        pltpu.VMEM((_CAPW,), jnp.int32),
        pltpu.VMEM((_CAPW,), jnp.int32),
        pltpu.VMEM((_CAP,), jnp.int32),
        pltpu.VMEM((_CAP,), jnp.int32),
        pltpu.VMEM((_CAP,), jnp.int32),
        pltpu.VMEM((_CAP,), jnp.int32),
        pltpu.VMEM((256,), jnp.int32),
        pltpu.VMEM((256,), jnp.int32),
        pltpu.VMEM((256,), jnp.int32),
        pltpu.VMEM((256,), jnp.float32),
        pltpu.VMEM((256,), jnp.float32),
        pltpu.VMEM((4096,), jnp.int32),
        pltpu.VMEM((256,), jnp.int32),
        pltpu.VMEM_SHARED((4096,), jnp.int32),
        pltpu.VMEM_SHARED((4096,), jnp.int32),
        pltpu.VMEM_SHARED((4096,), jnp.int32),
        pltpu.VMEM_SHARED((_CAP,), jnp.int32),
        pltpu.VMEM_SHARED((_CAP,), jnp.int32),
        pltpu.VMEM_SHARED((_CAP,), jnp.int32),
        pltpu.VMEM_SHARED((_CAP,), jnp.int32),
        pltpu.VMEM_SHARED((256,), jnp.int32),
        pltpu.VMEM_SHARED((256,), jnp.int32),
        pltpu.VMEM_SHARED((256,), jnp.int32),
        pltpu.VMEM_SHARED((256,), jnp.float32),
        pltpu.VMEM_SHARED((256,), jnp.float32),
    ],
)
def _select(bits_hbm, vols_hbm,
            w_hbm, lidx_hbm, sidx_hbm, lw_hbm, sw_hbm,
            abits_v, vols_v, wbuf_v, histA_v, histB_v,
            ckeyL_v, cidxL_v, ckeyS_v, cidxS_v,
            scatL_v, scatS_v,
            mkeyL_v, midxL_v, mkeyS_v, midxS_v,
            cnt_v, lidx_v, sidx_v, lw_v, sw_v,
            iota4k_v, iota256_v,
            histA_sh, histBL_sh, histBS_sh,
            ckeyL_sh, cidxL_sh, ckeyS_sh, cidxS_sh,
            cnt_sh, lidx_sh, sidx_sh, lw_sh, sw_sh):
    core = lax.axis_index("c")
    sid = lax.axis_index("s")

    @pl.when(core == 0)
    def _body():
        base = sid * _SL
        pltpu.sync_copy(bits_hbm.at[pl.ds(base, _SL)], abits_v)
        lanes = lax.iota(jnp.int32, 16)
        ones16 = jnp.ones((16,), jnp.int32)
        zero16f = jnp.zeros((16,), jnp.float32)
        zero16i = jnp.zeros((16,), jnp.int32)
        z = jnp.int32(0)

        def zero_hists(i, c):
            histA_v[pl.ds(i * 16, 16)] = zero16i
            histB_v[pl.ds(i * 16, 16)] = zero16i
            iota4k_v[pl.ds(i * 16, 16)] = i * 16 + lanes
            return c
        lax.fori_loop(0, 256, zero_hists, 0)

        def fill_iota256(i, c):
            iota256_v[pl.ds(i * 16, 16)] = i * 16 + lanes
            return c
        lax.fori_loop(0, 16, fill_iota256, 0)

        @pl.when(sid == 0)
        def _z():
            pltpu.sync_copy(vols_hbm, vols_v.at[pl.ds(0, _N)])
            pltpu.sync_copy(histA_v, histA_sh)
            pltpu.sync_copy(histA_v, histBL_sh)
            pltpu.sync_copy(histA_v, histBS_sh)

            def zero_misc(i, c):
                lidx_v[pl.ds(i * 16, 16)] = zero16i
                return c
            lax.fori_loop(0, 16, zero_misc, 0)
            pltpu.sync_copy(lidx_v, lidx_sh)
            pltpu.sync_copy(lidx_v, sidx_sh)
        plsc.subcore_barrier()

        def histo1(e, c):
            key = _keys_of(abits_v[pl.ds(e * 16, 16)])
            valid = (base + e * 16 + lanes) < _N
            bin1 = lax.shift_right_arithmetic(key, 24) + 128
            plsc.addupdate_scatter(histA_v, [bin1 * 16 + lanes], ones16,
                                   mask=valid)
            return c
        lax.fori_loop(0, _NVW, histo1, 0)
        pltpu.sync_copy(histA_v, histA_sh.at[iota4k_v], add=True)
        plsc.subcore_barrier()

        pltpu.sync_copy(histA_sh, histA_v)

        def scan_hi(t, carry):
            acc, binv, nab = carry
            b = 255 - t
            tot = jnp.sum(histA_v[pl.ds(b * 16, 16)])
            acc2 = acc + tot
            hit = (acc2 >= _K) & (acc < _K)
            return (acc2,
                    jnp.where(hit, b, binv),
                    jnp.where(hit, acc, nab))

        def scan_lo(t, carry):
            acc, binv, nbl = carry
            tot = jnp.sum(histA_v[pl.ds(t * 16, 16)])
            acc2 = acc + tot
            hit = (acc2 >= _K) & (acc < _K)
            return (acc2,
                    jnp.where(hit, t, binv),
                    jnp.where(hit, acc, nbl))

        _, binL1, nabL1 = lax.fori_loop(0, 256, scan_hi, (z, z, z))
        _, binS1, nblS1 = lax.fori_loop(0, 256, scan_lo, (z, z, z))

        def zero_again(i, c):
            histA_v[pl.ds(i * 16, 16)] = zero16i
            histB_v[pl.ds(i * 16, 16)] = zero16i
            return c
        lax.fori_loop(0, 256, zero_again, 0)

        def histo2(e, c):
            key = _keys_of(abits_v[pl.ds(e * 16, 16)])
            valid = (base + e * 16 + lanes) < _N
            bin1 = lax.shift_right_arithmetic(key, 24) + 128
            bin2 = jnp.bitwise_and(lax.shift_right_arithmetic(key, 16), 255)
            hidx = bin2 * 16 + lanes
            plsc.addupdate_scatter(histA_v, [hidx], ones16,
                                   mask=valid & (bin1 == binL1))
            plsc.addupdate_scatter(histB_v, [hidx], ones16,
                                   mask=valid & (bin1 == binS1))
            return c
        lax.fori_loop(0, _NVW, histo2, 0)
        pltpu.sync_copy(histA_v, histBL_sh.at[iota4k_v], add=True)
        pltpu.sync_copy(histB_v, histBS_sh.at[iota4k_v], add=True)
        plsc.subcore_barrier()

        pltpu.sync_copy(histBL_sh, histA_v)
        pltpu.sync_copy(histBS_sh, histB_v)

        def scan2_hi(t, carry):
            acc, binv = carry
            b = 255 - t
            tot = jnp.sum(histA_v[pl.ds(b * 16, 16)])
            acc2 = acc + tot
            hit = (acc2 >= _K) & (acc < _K)
            return acc2, jnp.where(hit, b, binv)

        def scan2_lo(t, carry):
            acc, binv = carry
            tot = jnp.sum(histB_v[pl.ds(t * 16, 16)])
            acc2 = acc + tot
            hit = (acc2 >= _K) & (acc < _K)
            return acc2, jnp.where(hit, t, binv)

        _, binL2 = lax.fori_loop(0, 256, scan2_hi, (nabL1, z))
        _, binS2 = lax.fori_loop(0, 256, scan2_lo, (nblS1, z))

        thrL = jnp.bitwise_or(lax.shift_left(binL1 - 128, 8), binL2)
        thrS = jnp.bitwise_or(lax.shift_left(binS1 - 128, 8), binS2)

        def compact(e, carry):
            offL, offS = carry
            key = _keys_of(abits_v[pl.ds(e * 16, 16)])
            gidx = base + e * 16 + lanes
            valid = gidx < _N
            t16 = lax.shift_right_arithmetic(key, 16)
            mL = valid & (t16 >= thrL)
            mS = valid & (t16 <= thrS)
            oL = jnp.minimum(offL, _CAPW - 16)
            oS = jnp.minimum(offS, _CAPW - 16)
            plsc.store_compressed(ckeyL_v.at[pl.ds(oL, 16)], key, mask=mL)
            plsc.store_compressed(cidxL_v.at[pl.ds(oL, 16)], gidx, mask=mL)
            plsc.store_compressed(ckeyS_v.at[pl.ds(oS, 16)], key, mask=mS)
            plsc.store_compressed(cidxS_v.at[pl.ds(oS, 16)], gidx, mask=mS)
            return (offL + jnp.sum(mL.astype(jnp.int32)),
                    offS + jnp.sum(mS.astype(jnp.int32)))
        cL, cS = lax.fori_loop(0, _NVW, compact, (z, z))
        cL = jnp.minimum(cL, _CAPW)
        cS = jnp.minimum(cS, _CAPW)
        cnt_vec = jnp.where(lanes == 0, cL, jnp.where(lanes == 1, cS, 0))
        cnt_v[pl.ds(0, 16)] = cnt_vec
        pltpu.sync_copy(cnt_v.at[pl.ds(0, 16)],
                        cnt_sh.at[pl.ds(sid * 16, 16)])
        plsc.subcore_barrier()

        pltpu.sync_copy(cnt_sh, cnt_v)
        offL = z
        offS = z
        myoffL = z
        myoffS = z
        for w in range(_NW):
            row = cnt_v[pl.ds(w * 16, 16)]
            cLw = row[0]
            cSw = row[1]
            myoffL = jnp.where(sid == w, offL, myoffL)
            myoffS = jnp.where(sid == w, offS, myoffS)
            offL = offL + cLw
            offS = offS + cSw
        nL = offL
        nS = offS

        def mkscat(e, c):
            ln = e * 16 + lanes
            tL = jnp.where(ln < cL, myoffL + ln, jnp.int32(_TRASH))
            tS = jnp.where(ln < cS, myoffS + ln, jnp.int32(_TRASH))
            scatL_v[pl.ds(e * 16, 16)] = tL
            scatS_v[pl.ds(e * 16, 16)] = tS
            return c
        lax.fori_loop(0, _CAPW // 16, mkscat, 0)
        pltpu.sync_copy(ckeyL_v, ckeyL_sh.at[scatL_v])
        pltpu.sync_copy(cidxL_v, cidxL_sh.at[scatL_v])
        pltpu.sync_copy(ckeyS_v, ckeyS_sh.at[scatS_v])
        pltpu.sync_copy(cidxS_v, cidxS_sh.at[scatS_v])
        plsc.subcore_barrier()

        pltpu.sync_copy(ckeyL_sh, mkeyL_v)
        pltpu.sync_copy(cidxL_sh, midxL_v)
        pltpu.sync_copy(ckeyS_sh, mkeyS_v)
        pltpu.sync_copy(cidxS_sh, midxS_v)

        def zero_out(i, c):
            lidx_v[pl.ds(i * 16, 16)] = zero16i
            sidx_v[pl.ds(i * 16, 16)] = zero16i
            return c
        lax.fori_loop(0, 16, zero_out, 0)

        def rank_side(ckey_v, cidx_v, n, rbase, idx_out_v):
            nvreg = lax.div(n + 15, jnp.int32(16))

            def do_iv(iv):
                kv = ckey_v[pl.ds(iv * 16, 16)]
                xv = cidx_v[pl.ds(iv * 16, 16)]

                def inner(jv, rank):
                    kjv = ckey_v[pl.ds(jv * 16, 16)]
                    ijv = cidx_v[pl.ds(jv * 16, 16)]
                    for l in range(16):
                        kj = kjv[l]
                        ij = ijv[l]
                        jok = (jv * 16 + l) < n
                        gt = ((kj > kv) | ((kj == kv) & (ij < xv))) & jok
                        rank = rank + gt.astype(jnp.int32)
                    return rank
                rank = lax.fori_loop(0, nvreg, inner, zero16i)
                pos = rank - rbase
                msk = (pos >= 0) & (pos < _K) & ((iv * 16 + lanes) < n)
                plsc.store_scatter(idx_out_v, [pos], xv, mask=msk)

            @pl.when(sid * 16 < n)
            def _a():
                do_iv(sid)

            @pl.when((sid + _NW) * 16 < n)
            def _b():
                do_iv(sid + _NW)

        rank_side(mkeyL_v, midxL_v, nL, z, lidx_v)
        rank_side(mkeyS_v, midxS_v, nS, nS - _K, sidx_v)
        pltpu.sync_copy(lidx_v, lidx_sh.at[iota256_v], add=True)
        pltpu.sync_copy(sidx_v, sidx_sh.at[iota256_v], add=True)
        plsc.subcore_barrier()

        @pl.when(sid == 0)
        def _fin():
            pltpu.sync_copy(lidx_sh, lidx_v)
            pltpu.sync_copy(sidx_sh, sidx_v)

            def weights_for(idx_v, w_v):
                def acc_loop(r, tot):
                    idxv = idx_v[pl.ds(r * 16, 16)]
                    volv = plsc.load_gather(vols_v, [idxv])
                    wv = 1.0 / (volv + _EPS)
                    wv = jnp.where((r * 16 + lanes) < _K, wv, 0.0)
                    w_v[pl.ds(r * 16, 16)] = wv
                    return tot + jnp.sum(wv)
                tot = lax.fori_loop(0, 13, acc_loop, jnp.float32(0.0))

                def norm_loop(r, c):
                    w_v[pl.ds(r * 16, 16)] = w_v[pl.ds(r * 16, 16)] / tot
                    return c
                lax.fori_loop(0, 13, norm_loop, 0)

            weights_for(lidx_v, lw_v)
            weights_for(sidx_v, sw_v)
            pltpu.sync_copy(lw_v, lw_sh)
            pltpu.sync_copy(sw_v, sw_sh)
            pltpu.sync_copy(lidx_v, lidx_hbm)
            pltpu.sync_copy(sidx_v, sidx_hbm)
            pltpu.sync_copy(lw_v, lw_hbm)
            pltpu.sync_copy(sw_v, sw_hbm)
        plsc.subcore_barrier()

        pltpu.sync_copy(lidx_sh, lidx_v)
        pltpu.sync_copy(sidx_sh, sidx_v)
        pltpu.sync_copy(lw_sh, lw_v)
        pltpu.sync_copy(sw_sh, sw_v)

        def zero_slice(i, c):
            wbuf_v[pl.ds(i * 16, 16)] = zero16f
            return c
        lax.fori_loop(0, _SL // 16, zero_slice, 0)

        def scat_side(idx_v, w_v, sign):
            def sloop(r, c):
                idxv = idx_v[pl.ds(r * 16, 16)]
                wv = w_v[pl.ds(r * 16, 16)]
                loc = idxv - base
                msk = ((r * 16 + lanes) < _K) & (idxv >= base) & (loc < _SL)
                plsc.store_scatter(wbuf_v, [loc], sign * wv, mask=msk)
                return c
            lax.fori_loop(0, 13, sloop, 0)

        scat_side(lidx_v, lw_v, jnp.float32(1.0))
        scat_side(sidx_v, sw_v, jnp.float32(-1.0))

        @pl.when(sid < _NW - 1)
        def _wfull():
            pltpu.sync_copy(wbuf_v.at[pl.ds(0, _SL)],
                            w_hbm.at[pl.ds(base, _SL)])

        @pl.when(sid == _NW - 1)
        def _wlast():
            pltpu.sync_copy(wbuf_v.at[pl.ds(0, _N - (_NW - 1) * _SL)],
                            w_hbm.at[pl.ds(base, _N - (_NW - 1) * _SL)])


def kernel(h, volatilities, W, b):
    w_pad_t = jnp.zeros((128, _D), jnp.float32).at[0, :].set(W[0])
    alpha, bits = _alpha_matvec(h, w_pad_t, b)
    weights, lidx, sidx, lw, sw = _select(bits.reshape(_NP), volatilities)
    return (alpha.reshape(_N), weights,
            lidx[:_K], sidx[:_K], lw[:_K], sw[:_K])
